# baseline (speedup 1.0000x reference)
import jax
import jax.numpy as jnp
from jax.experimental import pallas as pl
from jax.experimental.pallas import tpu as pltpu

_E = 64
_H = 1024
_I = 768
_T = 2048
_TM = 32
_NT = _T // _TM + _E


def _routing_kernel(e_ref, te_ref, ppos_ref):
    e = e_ref[...]
    lane = jax.lax.broadcasted_iota(jnp.int32, (_T, _E), 1)
    onehot = (e == lane).astype(jnp.int32)
    c = onehot
    s = 1
    while s < _T:
        shifted = jnp.concatenate(
            [jnp.zeros((s, _E), jnp.int32), c[:-s, :]], axis=0)
        c = c + shifted
        s *= 2
    counts = c[_T - 1:_T, :]
    rank = jnp.sum((c - onehot) * onehot, axis=1, keepdims=True)
    tiles_per_e = (counts + _TM - 1) // _TM
    ji = jax.lax.broadcasted_iota(jnp.int32, (_E, _E), 1)
    jj = jax.lax.broadcasted_iota(jnp.int32, (_E, _E), 0)
    tile_start = jnp.sum(jnp.where(ji < jj, tiles_per_e, 0),
                         axis=1, keepdims=True)
    used = tile_start[_E - 1, 0] + tiles_per_e[0, _E - 1]
    ti = jax.lax.broadcasted_iota(jnp.int32, (_NT, _E), 0)
    ts_row = jnp.reshape(tile_start, (1, _E))
    te = jnp.sum((ts_row <= ti).astype(jnp.int32), axis=1,
                 keepdims=True) - 1
    te = jnp.clip(te, 0, _E - 1)
    nt_i = jax.lax.broadcasted_iota(jnp.int32, (_NT, 1), 0)
    te_last = jnp.sum(jnp.where(nt_i == used - 1, te, 0))
    te_ref[...] = jnp.where(nt_i < used, te, te_last)
    ts_tok = jnp.sum(onehot * ts_row, axis=1, keepdims=True)
    ppos_ref[...] = ts_tok * _TM + rank


def _moe_kernel(tile_expert_ref, row_ids_ref, hs_ref, tw_ref, gu_ref, dp_ref,
                out_ref):
    i = pl.program_id(0)

    @pl.when(row_ids_ref[i * _TM] < _T)
    def _():
        xs = []
        ws = []
        for j in range(_TM):
            rid = row_ids_ref[i * _TM + j]
            src = jnp.minimum(rid, _T - 1)
            xs.append(hs_ref[pl.ds(src, 1), :])
            ws.append(tw_ref[pl.ds(src, 1), :])
        x = jnp.concatenate(xs, axis=0)
        w = jnp.concatenate(ws, axis=0)
        xb = x.astype(jnp.bfloat16)
        gu = jax.lax.dot_general(xb, gu_ref[0].astype(jnp.bfloat16),
                                 (((1,), (1,)), ((), ())),
                                 preferred_element_type=jnp.float32)
        gate = gu[:, :_I]
        up = gu[:, _I:]
        h = (gate * jax.nn.sigmoid(gate)) * up * w
        out = jax.lax.dot_general(h.astype(jnp.bfloat16),
                                  dp_ref[0].astype(jnp.bfloat16),
                                  (((1,), (1,)), ((), ())),
                                  preferred_element_type=jnp.float32)
        for j in range(_TM):
            rid = row_ids_ref[i * _TM + j]
            out_ref[pl.ds(rid, 1), :] = out[j:j + 1, :]


def kernel(hidden_states, top_k_index, top_k_weights, gate_up_proj, down_proj):
    e = top_k_index[:, :1].astype(jnp.int32)
    te, ppos = pl.pallas_call(
        _routing_kernel,
        out_shape=(jax.ShapeDtypeStruct((_NT, 1), jnp.int32),
                   jax.ShapeDtypeStruct((_T, 1), jnp.int32)),
    )(e)
    te = te[:, 0]
    row_ids = jnp.full((_NT * _TM,), _T, jnp.int32).at[ppos[:, 0]].set(
        jnp.arange(_T, dtype=jnp.int32))

    grid_spec = pltpu.PrefetchScalarGridSpec(
        num_scalar_prefetch=2,
        grid=(_NT,),
        in_specs=[
            pl.BlockSpec((_T, _H), lambda i, te_r, ri_r: (0, 0)),
            pl.BlockSpec((_T, 1), lambda i, te_r, ri_r: (0, 0)),
            pl.BlockSpec((1, 2 * _I, _H), lambda i, te_r, ri_r: (te_r[i], 0, 0)),
            pl.BlockSpec((1, _H, _I), lambda i, te_r, ri_r: (te_r[i], 0, 0)),
        ],
        out_specs=pl.BlockSpec((_T + 8, _H), lambda i, te_r, ri_r: (0, 0)),
    )
    out = pl.pallas_call(
        _moe_kernel,
        grid_spec=grid_spec,
        out_shape=jax.ShapeDtypeStruct((_T + 8, _H), jnp.float32),
    )(te, row_ids, hidden_states, top_k_weights, gate_up_proj, down_proj)
    return out[:_T]


# speedup vs baseline: 1.3841x; 1.3841x over previous
import jax
import jax.numpy as jnp
from jax.experimental import pallas as pl
from jax.experimental.pallas import tpu as pltpu

_E = 64
_H = 1024
_I = 768
_T = 2048
_TM = 64
_NT = _T // _TM + _E


def _routing_kernel(e_ref, te_ref, ppos_ref):
    e = e_ref[...]
    lane = jax.lax.broadcasted_iota(jnp.int32, (_T, _E), 1)
    onehot = (e == lane).astype(jnp.int32)
    c = onehot
    s = 1
    while s < _T:
        shifted = jnp.concatenate(
            [jnp.zeros((s, _E), jnp.int32), c[:-s, :]], axis=0)
        c = c + shifted
        s *= 2
    counts = c[_T - 1:_T, :]
    rank = jnp.sum((c - onehot) * onehot, axis=1, keepdims=True)
    tiles_per_e = (counts + _TM - 1) // _TM
    ji = jax.lax.broadcasted_iota(jnp.int32, (_E, _E), 1)
    jj = jax.lax.broadcasted_iota(jnp.int32, (_E, _E), 0)
    tile_start = jnp.sum(jnp.where(ji < jj, tiles_per_e, 0),
                         axis=1, keepdims=True)
    used = tile_start[_E - 1, 0] + tiles_per_e[0, _E - 1]
    ti = jax.lax.broadcasted_iota(jnp.int32, (_NT, _E), 0)
    ts_row = jnp.reshape(tile_start, (1, _E))
    te = jnp.sum((ts_row <= ti).astype(jnp.int32), axis=1,
                 keepdims=True) - 1
    te = jnp.clip(te, 0, _E - 1)
    nt_i = jax.lax.broadcasted_iota(jnp.int32, (_NT, 1), 0)
    te_last = jnp.sum(jnp.where(nt_i == used - 1, te, 0))
    te_ref[...] = jnp.where(nt_i < used, te, te_last)
    ts_tok = jnp.sum(onehot * ts_row, axis=1, keepdims=True)
    ppos_ref[...] = ts_tok * _TM + rank


def _moe_kernel(tile_expert_ref, row_ids_ref, hs_ref, tw_ref, gu_ref, dp_ref,
                out_ref):
    i = pl.program_id(0)

    @pl.when(row_ids_ref[i * _TM] < _T)
    def _():
        xs = []
        ws = []
        for j in range(_TM):
            rid = row_ids_ref[i * _TM + j]
            src = jnp.minimum(rid, _T - 1)
            xs.append(hs_ref[pl.ds(src, 1), :])
            ws.append(tw_ref[pl.ds(src, 1), :])
        x = jnp.concatenate(xs, axis=0)
        w = jnp.concatenate(ws, axis=0)
        xb = x.astype(jnp.bfloat16)
        gu = jax.lax.dot_general(xb, gu_ref[0].astype(jnp.bfloat16),
                                 (((1,), (1,)), ((), ())),
                                 preferred_element_type=jnp.float32)
        gate = gu[:, :_I]
        up = gu[:, _I:]
        h = (gate * jax.nn.sigmoid(gate)) * up * w
        out = jax.lax.dot_general(h.astype(jnp.bfloat16),
                                  dp_ref[0].astype(jnp.bfloat16),
                                  (((1,), (1,)), ((), ())),
                                  preferred_element_type=jnp.float32)
        rid0 = row_ids_ref[i * _TM]
        for j in reversed(range(_TM)):
            rid = row_ids_ref[i * _TM + j]
            tgt = jnp.where(rid < _T, rid, rid0)
            out_ref[pl.ds(tgt, 1), :] = out[j:j + 1, :]


def kernel(hidden_states, top_k_index, top_k_weights, gate_up_proj, down_proj):
    e = top_k_index[:, :1].astype(jnp.int32)
    te, ppos = pl.pallas_call(
        _routing_kernel,
        out_shape=(jax.ShapeDtypeStruct((_NT, 1), jnp.int32),
                   jax.ShapeDtypeStruct((_T, 1), jnp.int32)),
    )(e)
    te = te[:, 0]
    row_ids = jnp.full((_NT * _TM,), _T, jnp.int32).at[ppos[:, 0]].set(
        jnp.arange(_T, dtype=jnp.int32))

    grid_spec = pltpu.PrefetchScalarGridSpec(
        num_scalar_prefetch=2,
        grid=(_NT,),
        in_specs=[
            pl.BlockSpec((_T, _H), lambda i, te_r, ri_r: (0, 0)),
            pl.BlockSpec((_T, 1), lambda i, te_r, ri_r: (0, 0)),
            pl.BlockSpec((1, 2 * _I, _H), lambda i, te_r, ri_r: (te_r[i], 0, 0)),
            pl.BlockSpec((1, _H, _I), lambda i, te_r, ri_r: (te_r[i], 0, 0)),
        ],
        out_specs=pl.BlockSpec((_T, _H), lambda i, te_r, ri_r: (0, 0)),
    )
    return pl.pallas_call(
        _moe_kernel,
        grid_spec=grid_spec,
        out_shape=jax.ShapeDtypeStruct((_T, _H), jnp.float32),
    )(te, row_ids, hidden_states, top_k_weights, gate_up_proj, down_proj)
